# baseline (speedup 1.0000x reference)
import functools

import jax
import jax.numpy as jnp
from jax import lax
from jax.experimental import pallas as pl
from jax.experimental.pallas import tpu as pltpu
from jax.experimental.pallas import tpu_sc as plsc

_C = 19
_B = 8
_H = 512
_W = 512
_EPS = 1e-6
_HB = 64
_RC = 8

_NC = 2
_NS = 16
_NW = _NC * _NS
_NPIX = _B * _H * _W
_CHUNK = _NPIX // _NW
_VPC = _CHUNK // 16
_BINS = 512
_UNROLL = 16


_ROWS = _H // (_NW // _B)


def _sc_hist_body(t_hbm, out_hbm, tbuf, b0, b1, b2, b3):
    wid = lax.axis_index("s") * _NC + lax.axis_index("c")
    sample = wid % _B
    quarter = wid // _B
    pltpu.sync_copy(t_hbm.at[sample, pl.ds(quarter * _ROWS, _ROWS)], tbuf)

    banks = (b0, b1, b2, b3)
    zeros16 = jnp.zeros((16,), jnp.float32)

    def zbody(i, carry):
        s = pl.ds(i * 16, 16)
        for bank in banks:
            bank[s] = zeros16
        return carry

    lax.fori_loop(0, _BINS // 16, zbody, 0)

    lane32 = lax.iota(jnp.int32, 16) * 32
    ones16 = jnp.ones((16,), jnp.float32)

    def body(i, carry):
        r = i >> 1
        j0 = (i & 1) * (_W // 2)
        for j in range(_W // 32):
            t = tbuf[r, pl.ds(j0 + j * 16, 16)]
            t = jnp.minimum(jnp.maximum(t, 0), 31)
            plsc.addupdate_scatter(banks[j % 4], [lane32 + t], ones16)
        return carry

    lax.fori_loop(0, _ROWS * 2, body, 0)

    def cbody(i, carry):
        s = pl.ds(i * 16, 16)
        b0[s] = b0[s] + b1[s] + b2[s] + b3[s]
        return carry

    lax.fori_loop(0, _BINS // 16, cbody, 0)
    pltpu.sync_copy(b0, out_hbm.at[wid])


@functools.partial(
    pl.kernel,
    mesh=plsc.VectorSubcoreMesh(core_axis_name="c", subcore_axis_name="s"),
    out_type=jax.ShapeDtypeStruct((_NW, _BINS), jnp.float32),
    scratch_types=[
        pltpu.VMEM((_ROWS, _W), jnp.int32),
        pltpu.VMEM((_BINS,), jnp.float32),
        pltpu.VMEM((_BINS,), jnp.float32),
        pltpu.VMEM((_BINS,), jnp.float32),
        pltpu.VMEM((_BINS,), jnp.float32),
    ],
    compiler_params=pltpu.CompilerParams(
        needs_layout_passes=False, use_tc_tiling_on_sc=True
    ),
)
def _sc_hist(t_hbm, out_hbm, tbuf, b0, b1, b2, b3):
    _sc_hist_body(t_hbm, out_hbm, tbuf, b0, b1, b2, b3)


def _tc_softmax_body(logits_ref, out_ref, acc):
    b = pl.program_id(0)
    h = pl.program_id(1)

    @pl.when((b == 0) & (h == 0))
    def _init():
        out_ref[...] = jnp.zeros_like(out_ref)

    @pl.when(h == 0)
    def _reset():
        acc[...] = jnp.zeros_like(acc)

    for k in range(_HB // _RC):
        x = logits_ref[0, :, pl.ds(k * _RC, _RC), :]
        e = jnp.exp(x)
        denom = jnp.sum(e, axis=0, keepdims=True)
        acc[...] += e * (1.0 / denom)

    @pl.when(h == pl.num_programs(1) - 1)
    def _flush():
        psum = jnp.sum(acc[...], axis=(1, 2))
        rows = lax.broadcasted_iota(jnp.int32, (_B, _C), 0)
        sel = (rows == b).astype(jnp.float32)
        out_ref[...] += sel * psum[None, :]


def _tc_kl_body(pred_ref, hist_ref, out_ref):
    ph = hist_ref[...]
    hs = ph[:, 0:32]
    for l in range(1, 16):
        hs = hs + ph[:, l * 32:(l + 1) * 32]
    hq = hs[0:8] + hs[8:16] + hs[16:24] + hs[24:32]
    cnt = hq[:, :_C]
    ps = pred_ref[...]

    cols = lax.broadcasted_iota(jnp.int32, (_B, _C), 1)
    mask = ((cols != 0) & (cols != 1)).astype(jnp.float32)
    th = cnt * mask
    ps = ps * mask
    tt = jnp.sum(th, axis=1, keepdims=True)
    pt = jnp.sum(ps, axis=1, keepdims=True)
    td = th / (tt + _EPS)
    pd = ps / (pt + _EPS)
    kl = jnp.sum(td * (jnp.log(td + _EPS) - jnp.log(pd + _EPS)), axis=1,
                 keepdims=True)
    valid = ((tt > 0.0) & (pt > 0.0)).astype(jnp.float32)
    nv = jnp.sum(valid)
    loss = jnp.where(nv > 0.0, jnp.sum(kl * valid) / jnp.maximum(nv, 1.0), 0.0)
    out_ref[...] = jnp.broadcast_to(loss, (1, 1))


def kernel(logits, targets):
    hist = _sc_hist(targets)

    pred = pl.pallas_call(
        _tc_softmax_body,
        grid=(_B, _H // _HB),
        in_specs=[pl.BlockSpec((1, _C, _HB, _W), lambda b, h: (b, 0, h, 0))],
        out_specs=pl.BlockSpec((_B, _C), lambda b, h: (0, 0)),
        out_shape=jax.ShapeDtypeStruct((_B, _C), jnp.float32),
        scratch_shapes=[pltpu.VMEM((_C, _RC, _W), jnp.float32)],
    )(logits)

    out = pl.pallas_call(
        _tc_kl_body,
        in_specs=[
            pl.BlockSpec((_B, _C), lambda: (0, 0)),
            pl.BlockSpec((_NW, _BINS), lambda: (0, 0)),
        ],
        out_specs=pl.BlockSpec((1, 1), lambda: (0, 0)),
        out_shape=jax.ShapeDtypeStruct((1, 1), jnp.float32),
    )(pred, hist)
    return out[0, 0]


# speedup vs baseline: 1.1861x; 1.1861x over previous
import functools

import jax
import jax.numpy as jnp
from jax import lax
from jax.experimental import pallas as pl
from jax.experimental.pallas import tpu as pltpu
from jax.experimental.pallas import tpu_sc as plsc

_C = 19
_B = 8
_H = 512
_W = 512
_EPS = 1e-6
_HB = 128
_RC = 8

_NC = 2
_NS = 16
_NW = _NC * _NS
_NPIX = _B * _H * _W
_CHUNK = _NPIX // _NW
_VPC = _CHUNK // 16
_BINS = 512
_UNROLL = 16


_ROWS = _H // (_NW // _B)


def _sc_hist_body(t_hbm, out_hbm, tbuf, b0, b1, b2, b3):
    wid = lax.axis_index("s") * _NC + lax.axis_index("c")
    sample = wid % _B
    quarter = wid // _B
    pltpu.sync_copy(t_hbm.at[sample, pl.ds(quarter * _ROWS, _ROWS)], tbuf)

    banks = (b0, b1, b2, b3)
    zeros16 = jnp.zeros((16,), jnp.float32)

    def zbody(i, carry):
        s = pl.ds(i * 16, 16)
        for bank in banks:
            bank[s] = zeros16
        return carry

    lax.fori_loop(0, _BINS // 16, zbody, 0)

    lane32 = lax.iota(jnp.int32, 16) * 32
    ones16 = jnp.ones((16,), jnp.float32)

    def body(i, carry):
        r = i >> 1
        j0 = (i & 1) * (_W // 2)
        for j in range(_W // 32):
            t = tbuf[r, pl.ds(j0 + j * 16, 16)]
            t = jnp.minimum(jnp.maximum(t, 0), 31)
            plsc.addupdate_scatter(banks[j % 4], [lane32 + t], ones16)
        return carry

    lax.fori_loop(0, _ROWS * 2, body, 0)

    def cbody(i, carry):
        s = pl.ds(i * 16, 16)
        b0[s] = b0[s] + b1[s] + b2[s] + b3[s]
        return carry

    lax.fori_loop(0, _BINS // 16, cbody, 0)
    pltpu.sync_copy(b0, out_hbm.at[wid])


@functools.partial(
    pl.kernel,
    mesh=plsc.VectorSubcoreMesh(core_axis_name="c", subcore_axis_name="s"),
    out_type=jax.ShapeDtypeStruct((_NW, _BINS), jnp.float32),
    scratch_types=[
        pltpu.VMEM((_ROWS, _W), jnp.int32),
        pltpu.VMEM((_BINS,), jnp.float32),
        pltpu.VMEM((_BINS,), jnp.float32),
        pltpu.VMEM((_BINS,), jnp.float32),
        pltpu.VMEM((_BINS,), jnp.float32),
    ],
    compiler_params=pltpu.CompilerParams(
        needs_layout_passes=False, use_tc_tiling_on_sc=True
    ),
)
def _sc_hist(t_hbm, out_hbm, tbuf, b0, b1, b2, b3):
    _sc_hist_body(t_hbm, out_hbm, tbuf, b0, b1, b2, b3)


def _tc_softmax_body(logits_ref, out_ref, acc):
    b = pl.program_id(0)
    h = pl.program_id(1)

    @pl.when((b == 0) & (h == 0))
    def _init():
        out_ref[...] = jnp.zeros_like(out_ref)

    @pl.when(h == 0)
    def _reset():
        acc[...] = jnp.zeros_like(acc)

    for k in range(_HB // _RC):
        x = logits_ref[0, :, pl.ds(k * _RC, _RC), :]
        e = jnp.exp(x)
        denom = jnp.sum(e, axis=0, keepdims=True)
        acc[...] += e * (1.0 / denom)

    @pl.when(h == pl.num_programs(1) - 1)
    def _flush():
        psum = jnp.sum(acc[...], axis=(1, 2))
        rows = lax.broadcasted_iota(jnp.int32, (_B, _C), 0)
        sel = (rows == b).astype(jnp.float32)
        out_ref[...] += sel * psum[None, :]


def _tc_kl_body(pred_ref, hist_ref, out_ref):
    ph = hist_ref[...]
    hs = ph[:, 0:32]
    for l in range(1, 16):
        hs = hs + ph[:, l * 32:(l + 1) * 32]
    hq = hs[0:8] + hs[8:16] + hs[16:24] + hs[24:32]
    cnt = hq[:, :_C]
    ps = pred_ref[...]

    cols = lax.broadcasted_iota(jnp.int32, (_B, _C), 1)
    mask = ((cols != 0) & (cols != 1)).astype(jnp.float32)
    th = cnt * mask
    ps = ps * mask
    tt = jnp.sum(th, axis=1, keepdims=True)
    pt = jnp.sum(ps, axis=1, keepdims=True)
    td = th / (tt + _EPS)
    pd = ps / (pt + _EPS)
    kl = jnp.sum(td * (jnp.log(td + _EPS) - jnp.log(pd + _EPS)), axis=1,
                 keepdims=True)
    valid = ((tt > 0.0) & (pt > 0.0)).astype(jnp.float32)
    nv = jnp.sum(valid)
    loss = jnp.where(nv > 0.0, jnp.sum(kl * valid) / jnp.maximum(nv, 1.0), 0.0)
    out_ref[...] = jnp.broadcast_to(loss, (1, 1))


def kernel(logits, targets):
    hist = _sc_hist(targets)

    pred = pl.pallas_call(
        _tc_softmax_body,
        grid=(_B, _H // _HB),
        in_specs=[pl.BlockSpec((1, _C, _HB, _W), lambda b, h: (b, 0, h, 0))],
        out_specs=pl.BlockSpec((_B, _C), lambda b, h: (0, 0)),
        out_shape=jax.ShapeDtypeStruct((_B, _C), jnp.float32),
        scratch_shapes=[pltpu.VMEM((_C, _RC, _W), jnp.float32)],
    )(logits)

    out = pl.pallas_call(
        _tc_kl_body,
        in_specs=[
            pl.BlockSpec((_B, _C), lambda: (0, 0)),
            pl.BlockSpec((_NW, _BINS), lambda: (0, 0)),
        ],
        out_specs=pl.BlockSpec((1, 1), lambda: (0, 0)),
        out_shape=jax.ShapeDtypeStruct((1, 1), jnp.float32),
    )(pred, hist)
    return out[0, 0]


# speedup vs baseline: 1.3311x; 1.1222x over previous
import functools

import jax
import jax.numpy as jnp
from jax import lax
from jax.experimental import pallas as pl
from jax.experimental.pallas import tpu as pltpu
from jax.experimental.pallas import tpu_sc as plsc

_C = 19
_B = 8
_H = 512
_W = 512
_EPS = 1e-6
_HB = 256
_RC = 8

_NC = 2
_NS = 16
_NW = _NC * _NS
_NPIX = _B * _H * _W
_CHUNK = _NPIX // _NW
_VPC = _CHUNK // 16
_BINS = 512
_UNROLL = 16


_ROWS = _H // (_NW // _B)


def _sc_hist_body(t_hbm, out_hbm, tbuf, b0, b1, b2, b3):
    wid = lax.axis_index("s") * _NC + lax.axis_index("c")
    sample = wid % _B
    quarter = wid // _B
    pltpu.sync_copy(t_hbm.at[sample, pl.ds(quarter * _ROWS, _ROWS)], tbuf)

    banks = (b0, b1, b2, b3)
    zeros16 = jnp.zeros((16,), jnp.float32)

    def zbody(i, carry):
        s = pl.ds(i * 16, 16)
        for bank in banks:
            bank[s] = zeros16
        return carry

    lax.fori_loop(0, _BINS // 16, zbody, 0)

    lane32 = lax.iota(jnp.int32, 16) * 32
    ones16 = jnp.ones((16,), jnp.float32)

    def body(i, carry):
        r = i >> 1
        j0 = (i & 1) * (_W // 2)
        for j in range(_W // 32):
            t = tbuf[r, pl.ds(j0 + j * 16, 16)]
            t = jnp.minimum(jnp.maximum(t, 0), 31)
            plsc.addupdate_scatter(banks[j % 4], [lane32 + t], ones16)
        return carry

    lax.fori_loop(0, _ROWS * 2, body, 0)

    def cbody(i, carry):
        s = pl.ds(i * 16, 16)
        b0[s] = b0[s] + b1[s] + b2[s] + b3[s]
        return carry

    lax.fori_loop(0, _BINS // 16, cbody, 0)
    pltpu.sync_copy(b0, out_hbm.at[wid])


@functools.partial(
    pl.kernel,
    mesh=plsc.VectorSubcoreMesh(core_axis_name="c", subcore_axis_name="s"),
    out_type=jax.ShapeDtypeStruct((_NW, _BINS), jnp.float32),
    scratch_types=[
        pltpu.VMEM((_ROWS, _W), jnp.int32),
        pltpu.VMEM((_BINS,), jnp.float32),
        pltpu.VMEM((_BINS,), jnp.float32),
        pltpu.VMEM((_BINS,), jnp.float32),
        pltpu.VMEM((_BINS,), jnp.float32),
    ],
    compiler_params=pltpu.CompilerParams(
        needs_layout_passes=False, use_tc_tiling_on_sc=True
    ),
)
def _sc_hist(t_hbm, out_hbm, tbuf, b0, b1, b2, b3):
    _sc_hist_body(t_hbm, out_hbm, tbuf, b0, b1, b2, b3)


def _tc_softmax_body(logits_ref, out_ref, acc):
    b = pl.program_id(0)
    h = pl.program_id(1)

    @pl.when((b == 0) & (h == 0))
    def _init():
        out_ref[...] = jnp.zeros_like(out_ref)

    @pl.when(h == 0)
    def _reset():
        acc[...] = jnp.zeros_like(acc)

    for k in range(_HB // _RC):
        x = logits_ref[0, :, pl.ds(k * _RC, _RC), :]
        e = jnp.exp(x)
        denom = jnp.sum(e, axis=0, keepdims=True)
        acc[...] += e * (1.0 / denom)

    @pl.when(h == pl.num_programs(1) - 1)
    def _flush():
        psum = jnp.sum(acc[...], axis=(1, 2))
        rows = lax.broadcasted_iota(jnp.int32, (_B, _C), 0)
        sel = (rows == b).astype(jnp.float32)
        out_ref[...] += sel * psum[None, :]


def _tc_kl_body(pred_ref, hist_ref, out_ref):
    ph = hist_ref[...]
    hs = ph[:, 0:32]
    for l in range(1, 16):
        hs = hs + ph[:, l * 32:(l + 1) * 32]
    hq = hs[0:8] + hs[8:16] + hs[16:24] + hs[24:32]
    cnt = hq[:, :_C]
    ps = pred_ref[...]

    cols = lax.broadcasted_iota(jnp.int32, (_B, _C), 1)
    mask = ((cols != 0) & (cols != 1)).astype(jnp.float32)
    th = cnt * mask
    ps = ps * mask
    tt = jnp.sum(th, axis=1, keepdims=True)
    pt = jnp.sum(ps, axis=1, keepdims=True)
    td = th / (tt + _EPS)
    pd = ps / (pt + _EPS)
    kl = jnp.sum(td * (jnp.log(td + _EPS) - jnp.log(pd + _EPS)), axis=1,
                 keepdims=True)
    valid = ((tt > 0.0) & (pt > 0.0)).astype(jnp.float32)
    nv = jnp.sum(valid)
    loss = jnp.where(nv > 0.0, jnp.sum(kl * valid) / jnp.maximum(nv, 1.0), 0.0)
    out_ref[...] = jnp.broadcast_to(loss, (1, 1))


def kernel(logits, targets):
    hist = _sc_hist(targets)

    pred = pl.pallas_call(
        _tc_softmax_body,
        grid=(_B, _H // _HB),
        in_specs=[pl.BlockSpec((1, _C, _HB, _W), lambda b, h: (b, 0, h, 0))],
        out_specs=pl.BlockSpec((_B, _C), lambda b, h: (0, 0)),
        out_shape=jax.ShapeDtypeStruct((_B, _C), jnp.float32),
        scratch_shapes=[pltpu.VMEM((_C, _RC, _W), jnp.float32)],
    )(logits)

    out = pl.pallas_call(
        _tc_kl_body,
        in_specs=[
            pl.BlockSpec((_B, _C), lambda: (0, 0)),
            pl.BlockSpec((_NW, _BINS), lambda: (0, 0)),
        ],
        out_specs=pl.BlockSpec((1, 1), lambda: (0, 0)),
        out_shape=jax.ShapeDtypeStruct((1, 1), jnp.float32),
    )(pred, hist)
    return out[0, 0]


# speedup vs baseline: 1.3569x; 1.0194x over previous
import functools

import jax
import jax.numpy as jnp
from jax import lax
from jax.experimental import pallas as pl
from jax.experimental.pallas import tpu as pltpu
from jax.experimental.pallas import tpu_sc as plsc

_C = 19
_B = 8
_H = 512
_W = 512
_EPS = 1e-6
_HB = 512
_RC = 8

_NC = 2
_NS = 16
_NW = _NC * _NS
_NPIX = _B * _H * _W
_CHUNK = _NPIX // _NW
_VPC = _CHUNK // 16
_BINS = 512
_UNROLL = 16


_ROWS = _H // (_NW // _B)


def _sc_hist_body(t_hbm, out_hbm, tbuf, b0, b1, b2, b3):
    wid = lax.axis_index("s") * _NC + lax.axis_index("c")
    sample = wid % _B
    quarter = wid // _B
    pltpu.sync_copy(t_hbm.at[sample, pl.ds(quarter * _ROWS, _ROWS)], tbuf)

    banks = (b0, b1, b2, b3)
    zeros16 = jnp.zeros((16,), jnp.float32)

    def zbody(i, carry):
        s = pl.ds(i * 16, 16)
        for bank in banks:
            bank[s] = zeros16
        return carry

    lax.fori_loop(0, _BINS // 16, zbody, 0)

    lane32 = lax.iota(jnp.int32, 16) * 32
    ones16 = jnp.ones((16,), jnp.float32)

    def body(i, carry):
        r = i >> 1
        j0 = (i & 1) * (_W // 2)
        for j in range(_W // 32):
            t = tbuf[r, pl.ds(j0 + j * 16, 16)]
            t = jnp.minimum(jnp.maximum(t, 0), 31)
            plsc.addupdate_scatter(banks[j % 4], [lane32 + t], ones16)
        return carry

    lax.fori_loop(0, _ROWS * 2, body, 0)

    def cbody(i, carry):
        s = pl.ds(i * 16, 16)
        b0[s] = b0[s] + b1[s] + b2[s] + b3[s]
        return carry

    lax.fori_loop(0, _BINS // 16, cbody, 0)
    pltpu.sync_copy(b0, out_hbm.at[wid])


@functools.partial(
    pl.kernel,
    mesh=plsc.VectorSubcoreMesh(core_axis_name="c", subcore_axis_name="s"),
    out_type=jax.ShapeDtypeStruct((_NW, _BINS), jnp.float32),
    scratch_types=[
        pltpu.VMEM((_ROWS, _W), jnp.int32),
        pltpu.VMEM((_BINS,), jnp.float32),
        pltpu.VMEM((_BINS,), jnp.float32),
        pltpu.VMEM((_BINS,), jnp.float32),
        pltpu.VMEM((_BINS,), jnp.float32),
    ],
    compiler_params=pltpu.CompilerParams(
        needs_layout_passes=False, use_tc_tiling_on_sc=True
    ),
)
def _sc_hist(t_hbm, out_hbm, tbuf, b0, b1, b2, b3):
    _sc_hist_body(t_hbm, out_hbm, tbuf, b0, b1, b2, b3)


def _tc_softmax_body(logits_ref, out_ref, acc):
    b = pl.program_id(0)
    h = pl.program_id(1)

    @pl.when((b == 0) & (h == 0))
    def _init():
        out_ref[...] = jnp.zeros_like(out_ref)

    @pl.when(h == 0)
    def _reset():
        acc[...] = jnp.zeros_like(acc)

    for k in range(_HB // _RC):
        x = logits_ref[0, :, pl.ds(k * _RC, _RC), :]
        e = jnp.exp(x)
        denom = jnp.sum(e, axis=0, keepdims=True)
        acc[...] += e * (1.0 / denom)

    @pl.when(h == pl.num_programs(1) - 1)
    def _flush():
        psum = jnp.sum(acc[...], axis=(1, 2))
        rows = lax.broadcasted_iota(jnp.int32, (_B, _C), 0)
        sel = (rows == b).astype(jnp.float32)
        out_ref[...] += sel * psum[None, :]


def _tc_kl_body(pred_ref, hist_ref, out_ref):
    ph = hist_ref[...]
    hs = ph[:, 0:32]
    for l in range(1, 16):
        hs = hs + ph[:, l * 32:(l + 1) * 32]
    hq = hs[0:8] + hs[8:16] + hs[16:24] + hs[24:32]
    cnt = hq[:, :_C]
    ps = pred_ref[...]

    cols = lax.broadcasted_iota(jnp.int32, (_B, _C), 1)
    mask = ((cols != 0) & (cols != 1)).astype(jnp.float32)
    th = cnt * mask
    ps = ps * mask
    tt = jnp.sum(th, axis=1, keepdims=True)
    pt = jnp.sum(ps, axis=1, keepdims=True)
    td = th / (tt + _EPS)
    pd = ps / (pt + _EPS)
    kl = jnp.sum(td * (jnp.log(td + _EPS) - jnp.log(pd + _EPS)), axis=1,
                 keepdims=True)
    valid = ((tt > 0.0) & (pt > 0.0)).astype(jnp.float32)
    nv = jnp.sum(valid)
    loss = jnp.where(nv > 0.0, jnp.sum(kl * valid) / jnp.maximum(nv, 1.0), 0.0)
    out_ref[...] = jnp.broadcast_to(loss, (1, 1))


def kernel(logits, targets):
    hist = _sc_hist(targets)

    pred = pl.pallas_call(
        _tc_softmax_body,
        grid=(_B, _H // _HB),
        in_specs=[pl.BlockSpec((1, _C, _HB, _W), lambda b, h: (b, 0, h, 0))],
        out_specs=pl.BlockSpec((_B, _C), lambda b, h: (0, 0)),
        out_shape=jax.ShapeDtypeStruct((_B, _C), jnp.float32),
        scratch_shapes=[pltpu.VMEM((_C, _RC, _W), jnp.float32)],
    )(logits)

    out = pl.pallas_call(
        _tc_kl_body,
        in_specs=[
            pl.BlockSpec((_B, _C), lambda: (0, 0)),
            pl.BlockSpec((_NW, _BINS), lambda: (0, 0)),
        ],
        out_specs=pl.BlockSpec((1, 1), lambda: (0, 0)),
        out_shape=jax.ShapeDtypeStruct((1, 1), jnp.float32),
    )(pred, hist)
    return out[0, 0]


# speedup vs baseline: 1.3575x; 1.0004x over previous
import functools

import jax
import jax.numpy as jnp
from jax import lax
from jax.experimental import pallas as pl
from jax.experimental.pallas import tpu as pltpu
from jax.experimental.pallas import tpu_sc as plsc

_C = 19
_B = 8
_H = 512
_W = 512
_EPS = 1e-6
_HB = 512
_RC = 8

_NC = 2
_NS = 16
_NW = _NC * _NS
_BINS = 512


_ROWS = _H // (_NW // _B)


def _sc_hist_body(t_hbm, out_hbm, tbuf, b0, b1, b2, b3):
    wid = lax.axis_index("s") * _NC + lax.axis_index("c")
    sample = wid % _B
    quarter = wid // _B
    pltpu.sync_copy(t_hbm.at[sample, pl.ds(quarter * _ROWS, _ROWS)], tbuf)

    banks = (b0, b1, b2, b3)
    zeros16 = jnp.zeros((16,), jnp.float32)

    def zbody(i, carry):
        s = pl.ds(i * 16, 16)
        for bank in banks:
            bank[s] = zeros16
        return carry

    lax.fori_loop(0, _BINS // 16, zbody, 0)

    lane32 = lax.iota(jnp.int32, 16) * 32
    ones16 = jnp.ones((16,), jnp.float32)

    def body(i, carry):
        r = i >> 1
        j0 = (i & 1) * (_W // 2)
        for j in range(_W // 32):
            t = tbuf[r, pl.ds(j0 + j * 16, 16)]
            t = jnp.minimum(jnp.maximum(t, 0), 31)
            plsc.addupdate_scatter(banks[j % 4], [lane32 + t], ones16)
        return carry

    lax.fori_loop(0, _ROWS * 2, body, 0)

    def cbody(i, carry):
        s = pl.ds(i * 16, 16)
        b0[s] = b0[s] + b1[s] + b2[s] + b3[s]
        return carry

    lax.fori_loop(0, _BINS // 16, cbody, 0)
    pltpu.sync_copy(b0, out_hbm.at[wid])


@functools.partial(
    pl.kernel,
    mesh=plsc.VectorSubcoreMesh(core_axis_name="c", subcore_axis_name="s"),
    out_type=jax.ShapeDtypeStruct((_NW, _BINS), jnp.float32),
    scratch_types=[
        pltpu.VMEM((_ROWS, _W), jnp.int32),
        pltpu.VMEM((_BINS,), jnp.float32),
        pltpu.VMEM((_BINS,), jnp.float32),
        pltpu.VMEM((_BINS,), jnp.float32),
        pltpu.VMEM((_BINS,), jnp.float32),
    ],
    compiler_params=pltpu.CompilerParams(
        needs_layout_passes=False, use_tc_tiling_on_sc=True
    ),
)
def _sc_hist(t_hbm, out_hbm, tbuf, b0, b1, b2, b3):
    _sc_hist_body(t_hbm, out_hbm, tbuf, b0, b1, b2, b3)


def _tc_softmax_body(logits_ref, out_ref, acc):
    b = pl.program_id(0)
    h = pl.program_id(1)

    @pl.when((b == 0) & (h == 0))
    def _init():
        out_ref[...] = jnp.zeros_like(out_ref)

    @pl.when(h == 0)
    def _reset():
        acc[...] = jnp.zeros_like(acc)

    for k in range(_HB // _RC):
        x = logits_ref[0, :, pl.ds(k * _RC, _RC), :]
        e = jnp.exp(x)
        denom = jnp.sum(e, axis=0, keepdims=True)
        acc[...] += e * (1.0 / denom)

    @pl.when(h == pl.num_programs(1) - 1)
    def _flush():
        psum = jnp.sum(acc[...], axis=(1, 2))
        rows = lax.broadcasted_iota(jnp.int32, (_B, _C), 0)
        sel = (rows == b).astype(jnp.float32)
        out_ref[...] += sel * psum[None, :]


def _tc_kl_body(pred_ref, hist_ref, out_ref):
    ph = hist_ref[...]
    hs = ph[:, 0:32]
    for l in range(1, 16):
        hs = hs + ph[:, l * 32:(l + 1) * 32]
    hq = hs[0:8] + hs[8:16] + hs[16:24] + hs[24:32]
    cnt = hq[:, :_C]
    ps = pred_ref[...]

    cols = lax.broadcasted_iota(jnp.int32, (_B, _C), 1)
    mask = ((cols != 0) & (cols != 1)).astype(jnp.float32)
    th = cnt * mask
    ps = ps * mask
    tt = jnp.sum(th, axis=1, keepdims=True)
    pt = jnp.sum(ps, axis=1, keepdims=True)
    td = th / (tt + _EPS)
    pd = ps / (pt + _EPS)
    kl = jnp.sum(td * (jnp.log(td + _EPS) - jnp.log(pd + _EPS)), axis=1,
                 keepdims=True)
    valid = ((tt > 0.0) & (pt > 0.0)).astype(jnp.float32)
    nv = jnp.sum(valid)
    loss = jnp.where(nv > 0.0, jnp.sum(kl * valid) / jnp.maximum(nv, 1.0), 0.0)
    out_ref[...] = jnp.broadcast_to(loss, (1, 1))


def kernel(logits, targets):
    hist = _sc_hist(targets)

    pred = pl.pallas_call(
        _tc_softmax_body,
        grid=(_B, _H // _HB),
        in_specs=[pl.BlockSpec((1, _C, _HB, _W), lambda b, h: (b, 0, h, 0))],
        out_specs=pl.BlockSpec((_B, _C), lambda b, h: (0, 0)),
        out_shape=jax.ShapeDtypeStruct((_B, _C), jnp.float32),
        scratch_shapes=[pltpu.VMEM((_C, _RC, _W), jnp.float32)],
    )(logits)

    out = pl.pallas_call(
        _tc_kl_body,
        in_specs=[
            pl.BlockSpec((_B, _C), lambda: (0, 0)),
            pl.BlockSpec((_NW, _BINS), lambda: (0, 0)),
        ],
        out_specs=pl.BlockSpec((1, 1), lambda: (0, 0)),
        out_shape=jax.ShapeDtypeStruct((1, 1), jnp.float32),
    )(pred, hist)
    return out[0, 0]
